# baseline (speedup 1.0000x reference)
import functools

import jax
import jax.numpy as jnp
from jax import lax
from jax.experimental import pallas as pl
from jax.experimental.pallas import tpu as pltpu

NUM_TOKENS = 32768
HIDDEN = 768
NUM_EXPERTS = 8
TOP_K = 2
NSTREAM = 16
SUB = 512
WAVE = NSTREAM * SUB


def _router_body(*refs):
    xrefs = refs[:NSTREAM]
    (w_ref, imp_ref, logits_ref, idx_ref, wts_ref,
     var_ref, ent_ref, load_acc, ent_acc) = refs[NSTREAM:]
    i = pl.program_id(0)
    nsteps = pl.num_programs(0)

    @pl.when(i == 0)
    def _init():
        load_acc[...] = jnp.zeros_like(load_acc)
        ent_acc[...] = jnp.zeros_like(ent_acc)

    imp = imp_ref[...]
    imp_m = jnp.max(imp, axis=0, keepdims=True)
    imp_e = jnp.exp(imp - imp_m)
    imp_w = imp_e / jnp.sum(imp_e, axis=0, keepdims=True)
    bias = jnp.log(imp_w + 1e-8)

    w = w_ref[...]
    load_sum = jnp.zeros((NUM_EXPERTS, 1), jnp.float32)
    ent_sum = jnp.float32(0.0)

    for k in range(NSTREAM):
        x = xrefs[k][...]
        logits = jax.lax.dot_general(
            x, w, (((1,), (1,)), ((), ())),
            preferred_element_type=jnp.float32)

        lt = logits.T + bias
        cols = pl.ds(k * SUB, SUB)
        logits_ref[:, cols] = lt

        iota = lax.broadcasted_iota(jnp.int32, lt.shape, 0).astype(jnp.float32)
        m1 = jnp.max(lt, axis=0, keepdims=True)
        i1 = jnp.min(jnp.where(lt == m1, iota, jnp.float32(NUM_EXPERTS)),
                     axis=0, keepdims=True)
        neg = jnp.float32(-3.0e38)
        masked = jnp.where(iota == i1, neg, lt)
        m2 = jnp.max(masked, axis=0, keepdims=True)
        i2 = jnp.min(jnp.where(masked == m2, iota, jnp.float32(NUM_EXPERTS)),
                     axis=0, keepdims=True)

        e2 = jnp.exp(m2 - m1)
        w1 = 1.0 / (1.0 + e2)
        w2 = 1.0 - w1

        idx_ref[:, cols] = jnp.concatenate([i1, i2], axis=0).astype(jnp.int32)
        wts_ref[:, cols] = jnp.concatenate([w1, w2], axis=0)

        p = jnp.exp(lt - m1)
        p = p / jnp.sum(p, axis=0, keepdims=True)
        load_sum = load_sum + jnp.sum(p, axis=1, keepdims=True)
        ent_sum = ent_sum - jnp.sum(p * jnp.log(p + 1e-8))

    load_acc[...] += load_sum
    ent_acc[...] += jnp.full((1, 1), ent_sum, jnp.float32)

    @pl.when(i == nsteps - 1)
    def _fin():
        load = load_acc[...] / jnp.float32(NUM_TOKENS)
        mean = jnp.sum(load) / jnp.float32(NUM_EXPERTS)
        var_ref[...] = jnp.full(
            (1, 1), jnp.sum((load - mean) ** 2) / jnp.float32(NUM_EXPERTS),
            jnp.float32)
        ent_ref[...] = ent_acc[...] / jnp.float32(NUM_TOKENS)


@functools.partial(jax.jit, static_argnames=())
def kernel(hidden_states, W, expert_importance):
    T, H = hidden_states.shape
    E = W.shape[0]
    out_shapes = (
        jax.ShapeDtypeStruct((E, T), jnp.float32),
        jax.ShapeDtypeStruct((TOP_K, T), jnp.int32),
        jax.ShapeDtypeStruct((TOP_K, T), jnp.float32),
        jax.ShapeDtypeStruct((1, 1), jnp.float32),
        jax.ShapeDtypeStruct((1, 1), jnp.float32),
    )

    def _xspec(k):
        return pl.BlockSpec((SUB, H), lambda i, k=k: (NSTREAM * i + k, 0))

    logits_t, idx_t, wts_t, var, ent = pl.pallas_call(
        _router_body,
        grid=(T // WAVE,),
        in_specs=[_xspec(k) for k in range(NSTREAM)] + [
            pl.BlockSpec((E, H), lambda i: (0, 0)),
            pl.BlockSpec((E, 1), lambda i: (0, 0)),
        ],
        out_specs=(
            pl.BlockSpec((E, WAVE), lambda i: (0, i)),
            pl.BlockSpec((TOP_K, WAVE), lambda i: (0, i)),
            pl.BlockSpec((TOP_K, WAVE), lambda i: (0, i)),
            pl.BlockSpec((1, 1), lambda i: (0, 0)),
            pl.BlockSpec((1, 1), lambda i: (0, 0)),
        ),
        out_shape=out_shapes,
        scratch_shapes=[
            pltpu.VMEM((E, 1), jnp.float32),
            pltpu.VMEM((1, 1), jnp.float32),
        ],
    )(*([hidden_states] * NSTREAM), W, expert_importance.reshape(E, 1))
    return (logits_t.T, idx_t.T, wts_t.T, var[0, 0], ent[0, 0])


# speedup vs baseline: 1.0138x; 1.0138x over previous
import functools

import jax
import jax.numpy as jnp
from jax import lax
from jax.experimental import pallas as pl
from jax.experimental.pallas import tpu as pltpu

NUM_TOKENS = 32768
HIDDEN = 768
NUM_EXPERTS = 8
TOP_K = 2
NSTREAM = 8
SUB = 1024
WAVE = NSTREAM * SUB


def _router_body(*refs):
    xrefs = refs[:NSTREAM]
    (w_ref, imp_ref, logits_ref, idx_ref, wts_ref,
     var_ref, ent_ref, load_acc, ent_acc) = refs[NSTREAM:]
    i = pl.program_id(0)
    nsteps = pl.num_programs(0)

    @pl.when(i == 0)
    def _init():
        load_acc[...] = jnp.zeros_like(load_acc)
        ent_acc[...] = jnp.zeros_like(ent_acc)

    imp = imp_ref[...]
    imp_m = jnp.max(imp, axis=0, keepdims=True)
    imp_e = jnp.exp(imp - imp_m)
    imp_w = imp_e / jnp.sum(imp_e, axis=0, keepdims=True)
    bias = jnp.log(imp_w + 1e-8)

    w = w_ref[...]
    load_sum = jnp.zeros((NUM_EXPERTS, 1), jnp.float32)
    ent_sum = jnp.float32(0.0)

    for k in range(NSTREAM):
        x = xrefs[k][...]
        logits = jax.lax.dot_general(
            x, w, (((1,), (1,)), ((), ())),
            preferred_element_type=jnp.float32)

        lt = logits.T + bias
        cols = pl.ds(k * SUB, SUB)
        logits_ref[:, cols] = lt

        iota = lax.broadcasted_iota(jnp.int32, lt.shape, 0).astype(jnp.float32)
        m1 = jnp.max(lt, axis=0, keepdims=True)
        i1 = jnp.min(jnp.where(lt == m1, iota, jnp.float32(NUM_EXPERTS)),
                     axis=0, keepdims=True)
        neg = jnp.float32(-3.0e38)
        masked = jnp.where(iota == i1, neg, lt)
        m2 = jnp.max(masked, axis=0, keepdims=True)
        i2 = jnp.min(jnp.where(masked == m2, iota, jnp.float32(NUM_EXPERTS)),
                     axis=0, keepdims=True)

        e2 = jnp.exp(m2 - m1)
        w1 = 1.0 / (1.0 + e2)
        w2 = 1.0 - w1

        idx_ref[:, cols] = jnp.concatenate([i1, i2], axis=0).astype(jnp.int32)
        wts_ref[:, cols] = jnp.concatenate([w1, w2], axis=0)

        p = jnp.exp(lt - m1)
        p = p / jnp.sum(p, axis=0, keepdims=True)
        load_sum = load_sum + jnp.sum(p, axis=1, keepdims=True)
        ent_sum = ent_sum - jnp.sum(p * jnp.log(p + 1e-8))

    load_acc[...] += load_sum
    ent_acc[...] += jnp.full((1, 1), ent_sum, jnp.float32)

    @pl.when(i == nsteps - 1)
    def _fin():
        load = load_acc[...] / jnp.float32(NUM_TOKENS)
        mean = jnp.sum(load) / jnp.float32(NUM_EXPERTS)
        var_ref[...] = jnp.full(
            (1, 1), jnp.sum((load - mean) ** 2) / jnp.float32(NUM_EXPERTS),
            jnp.float32)
        ent_ref[...] = ent_acc[...] / jnp.float32(NUM_TOKENS)


@functools.partial(jax.jit, static_argnames=())
def kernel(hidden_states, W, expert_importance):
    T, H = hidden_states.shape
    E = W.shape[0]
    out_shapes = (
        jax.ShapeDtypeStruct((E, T), jnp.float32),
        jax.ShapeDtypeStruct((TOP_K, T), jnp.int32),
        jax.ShapeDtypeStruct((TOP_K, T), jnp.float32),
        jax.ShapeDtypeStruct((1, 1), jnp.float32),
        jax.ShapeDtypeStruct((1, 1), jnp.float32),
    )

    def _xspec(k):
        return pl.BlockSpec((SUB, H), lambda i, k=k: (NSTREAM * i + k, 0))

    logits_t, idx_t, wts_t, var, ent = pl.pallas_call(
        _router_body,
        grid=(T // WAVE,),
        in_specs=[_xspec(k) for k in range(NSTREAM)] + [
            pl.BlockSpec((E, H), lambda i: (0, 0)),
            pl.BlockSpec((E, 1), lambda i: (0, 0)),
        ],
        out_specs=(
            pl.BlockSpec((E, WAVE), lambda i: (0, i)),
            pl.BlockSpec((TOP_K, WAVE), lambda i: (0, i)),
            pl.BlockSpec((TOP_K, WAVE), lambda i: (0, i)),
            pl.BlockSpec((1, 1), lambda i: (0, 0)),
            pl.BlockSpec((1, 1), lambda i: (0, 0)),
        ),
        out_shape=out_shapes,
        scratch_shapes=[
            pltpu.VMEM((E, 1), jnp.float32),
            pltpu.VMEM((1, 1), jnp.float32),
        ],
    )(*([hidden_states] * NSTREAM), W, expert_importance.reshape(E, 1))
    return (logits_t.T, idx_t.T, wts_t.T, var[0, 0], ent[0, 0])
